# Initial kernel scaffold; baseline (speedup 1.0000x reference)
import jax
import jax.numpy as jnp
from jax.experimental import pallas as pl


def kernel(points, normals, distances, w1a, b1a, w1b, b1b, w2a, b2a, w2b, b2b, w3a, b3a, w3b, b3b, w4, b4, w5, b5):
    raise NotImplementedError("write your pallas kernel here")


# speedup vs baseline: 4.9251x; 4.9251x over previous
import functools

import jax
import jax.numpy as jnp
from jax import lax
from jax.experimental import pallas as pl
from jax.experimental.pallas import tpu as pltpu
from jax.experimental.pallas import tpu_sc as plsc

_N = 10000
_K = 16
_P = 8
_THR = 0.05
_NPAD = 10240
_BQ = 256
_BN = 256
_GRID = _NPAD // _BQ
_F32 = jnp.float32

_NC = 2
_NS = 16
_NW = _NC * _NS
_GB = _K * _NPAD
_BPW = _GB // _NW
_CH = 512


def _knn_body(xq_ref, xt_ref, nbr_ref):
    i = pl.program_id(0)
    xq = xq_ref[...]
    xt = xt_ref[...]
    sqq = jnp.sum(xq * xq, axis=1, keepdims=True)
    sqt = jnp.sum(xt * xt, axis=0, keepdims=True)
    mm = jnp.dot(xq, xt, preferred_element_type=_F32)
    d = sqq + sqt - 2.0 * mm
    col = jax.lax.broadcasted_iota(jnp.int32, d.shape, 1)
    row = jax.lax.broadcasted_iota(jnp.int32, d.shape, 0) + i * _BQ
    inf = _F32(jnp.inf)
    d = jnp.where((col == row) | (col >= _N), inf, d)
    big = jnp.int32(2 ** 30)
    idxs = []
    for _ in range(_K):
        m = jnp.min(d, axis=1, keepdims=True)
        am = jnp.min(jnp.where(d == m, col, big), axis=1, keepdims=True)
        idxs.append(am)
        d = jnp.where(col == am, inf, d)
    nbr_ref[...] = jnp.concatenate(idxs, axis=1)


def _tab_body(x_ref, wa_ref, ba_ref, wb_ref, a_ref, b_ref):
    x = x_ref[...]
    a_ref[...] = jnp.dot(x, wa_ref[...], preferred_element_type=_F32) + ba_ref[...]
    bt = jnp.dot(x, wb_ref[...], preferred_element_type=_F32)
    b_ref[...] = jnp.concatenate([bt, jnp.zeros_like(bt)], axis=1)


def _sc_gather_body(tab_hbm, idx_hbm, out_hbm, idx_v, rows_v, sem):
    wid = lax.axis_index("s") * _NC + lax.axis_index("c")
    base = wid * _BPW

    def step(i, carry):
        off = base + i * _CH
        pltpu.sync_copy(idx_hbm.at[pl.ds(off, _CH)], idx_v)
        pltpu.async_copy(tab_hbm.at[idx_v], rows_v, sem).wait()
        pltpu.sync_copy(rows_v, out_hbm.at[pl.ds(off, _CH)])
        return carry

    lax.fori_loop(0, _BPW // _CH, step, 0)


def _sc_gather(btab, idx_flat):
    mesh = plsc.VectorSubcoreMesh(core_axis_name="c", subcore_axis_name="s")
    call = functools.partial(
        pl.kernel,
        mesh=mesh,
        out_type=jax.ShapeDtypeStruct((_GB, 128), _F32),
        scratch_types=[
            pltpu.VMEM((_CH,), jnp.int32),
            pltpu.VMEM((_CH, 128), _F32),
            pltpu.SemaphoreType.DMA,
        ],
    )(_sc_gather_body)
    return call(btab, idx_flat)


def _conv_body(a_ref, bg_ref, wb_ref, bb_ref, f_ref):
    a = a_ref[...]
    wb = wb_ref[...]
    acc = None
    for j in range(_K):
        bj = bg_ref[j][:, :64]
        h = jnp.dot(jnp.maximum(a + bj, 0.0), wb, preferred_element_type=_F32)
        acc = h if acc is None else jnp.maximum(acc, h)
    f_ref[...] = acc + bb_ref[...]


def _mlp_body(f1_ref, f2_ref, f3_ref, p_ref, w4_ref, b4_ref, w5_ref, b5_ref, o_ref):
    feat = jnp.concatenate([f1_ref[...], f2_ref[...], f3_ref[...]], axis=1)
    h = jnp.maximum(jnp.dot(feat, w4_ref[...], preferred_element_type=_F32) + b4_ref[...], 0.0)
    r = jnp.dot(h, w5_ref[...], preferred_element_type=_F32) + b5_ref[...]
    o_ref[...] = p_ref[...] + r


def _stats_body(pts_ref, nt_ref, dv_ref, s_ref):
    i = pl.program_id(0)
    pts = pts_ref[...]
    pd = jnp.dot(pts, nt_ref[...], preferred_element_type=_F32) - dv_ref[...]
    rowid = jax.lax.broadcasted_iota(jnp.int32, pd.shape, 0) + i * _BN
    mask = jnp.where((jnp.abs(pd) < _THR) & (rowid < _N), 1.0, 0.0)
    x = pts[:, 0:1]
    y = pts[:, 1:2]
    z = pts[:, 2:3]
    rows = [
        jnp.sum(mask, axis=0, keepdims=True),
        jnp.sum(mask * x, axis=0, keepdims=True),
        jnp.sum(mask * y, axis=0, keepdims=True),
        jnp.sum(mask * z, axis=0, keepdims=True),
        jnp.sum(mask * x * x, axis=0, keepdims=True),
        jnp.sum(mask * x * y, axis=0, keepdims=True),
        jnp.sum(mask * x * z, axis=0, keepdims=True),
        jnp.sum(mask * y * y, axis=0, keepdims=True),
        jnp.sum(mask * y * z, axis=0, keepdims=True),
        jnp.sum(mask * z * z, axis=0, keepdims=True),
    ]
    rows += [jnp.zeros((1, _P), _F32)] * 6
    contrib = jnp.concatenate(rows, axis=0)
    prev = jnp.where(i == 0, jnp.zeros_like(contrib), s_ref[...])
    s_ref[...] = prev + contrib


def _eig_smallest(cov, nrows):
    a = [[cov[0], cov[1], cov[2]],
         [cov[1], cov[3], cov[4]],
         [cov[2], cov[4], cov[5]]]
    one = jnp.ones((1, _P), _F32)
    zero = jnp.zeros((1, _P), _F32)
    v = [[one, zero, zero], [zero, one, zero], [zero, zero, one]]
    for _ in range(8):
        for (p, q) in ((0, 1), (0, 2), (1, 2)):
            apq = a[p][q]
            small = jnp.abs(apq) < 1e-30
            apq_s = jnp.where(small, 1.0, apq)
            tau = (a[q][q] - a[p][p]) / (2.0 * apq_s)
            t = jnp.sign(tau) / (jnp.abs(tau) + jnp.sqrt(1.0 + tau * tau))
            t = jnp.where(jnp.sign(tau) == 0.0, 1.0 / (jnp.abs(tau) + jnp.sqrt(1.0 + tau * tau)), t)
            t = jnp.where(small, 0.0, t)
            c = 1.0 / jnp.sqrt(1.0 + t * t)
            s = t * c
            app = a[p][p] - t * apq
            aqq = a[q][q] + t * apq
            k = 3 - p - q
            akp = c * a[k][p] - s * a[k][q]
            akq = s * a[k][p] + c * a[k][q]
            a[p][p] = app
            a[q][q] = aqq
            a[p][q] = zero
            a[q][p] = zero
            a[k][p] = akp
            a[p][k] = akp
            a[k][q] = akq
            a[q][k] = akq
            for r in range(3):
                vrp = c * v[r][p] - s * v[r][q]
                vrq = s * v[r][p] + c * v[r][q]
                v[r][p] = vrp
                v[r][q] = vrq
    l0, l1, l2 = a[0][0], a[1][1], a[2][2]
    is0 = (l0 <= l1) & (l0 <= l2)
    is1 = jnp.logical_not(l0 <= l1) & (l1 <= l2)
    rn = [jnp.where(is0, v[r][0], jnp.where(is1, v[r][1], v[r][2])) for r in range(3)]
    dt = rn[0] * nrows[0] + rn[1] * nrows[1] + rn[2] * nrows[2]
    sgn = jnp.where(dt < 0.0, -1.0, 1.0)
    return [rn[r] * sgn for r in range(3)]


def _proj_body(pts_ref, s_ref, nt_ref, dv_ref, o_ref):
    pts = pts_ref[...]
    st = s_ref[...]
    cnt = st[0:1, :]
    safe = jnp.maximum(cnt, 1.0)
    cx = st[1:2, :] / safe
    cy = st[2:3, :] / safe
    cz = st[3:4, :] / safe
    cov = [st[4:5, :] - cnt * cx * cx,
           st[5:6, :] - cnt * cx * cy,
           st[6:7, :] - cnt * cx * cz,
           st[7:8, :] - cnt * cy * cy,
           st[8:9, :] - cnt * cy * cz,
           st[9:10, :] - cnt * cz * cz]
    nt = nt_ref[...]
    nrows = [nt[0:1, :], nt[1:2, :], nt[2:3, :]]
    rn = _eig_smallest(cov, nrows)
    rd = cx * rn[0] + cy * rn[1] + cz * rn[2]
    valid = jnp.where(cnt >= 3.0, 1.0, 0.0)

    pd = jnp.dot(pts, nt, preferred_element_type=_F32) - dv_ref[...]
    mask = jnp.where(jnp.abs(pd) < _THR, 1.0, 0.0) * valid

    px = pts[:, 0:1]
    py = pts[:, 1:2]
    pz = pts[:, 2:3]
    for p in range(_P):
        w = mask[:, p:p + 1]
        rx = rn[0][0:1, p:p + 1]
        ry = rn[1][0:1, p:p + 1]
        rz = rn[2][0:1, p:p + 1]
        dot = px * rx + py * ry + pz * rz
        adj = w * (dot - rd[0:1, p:p + 1])
        px = px - adj * rx
        py = py - adj * ry
        pz = pz - adj * rz
    o_ref[...] = jnp.concatenate([px, py, pz, jnp.zeros((px.shape[0], 5), _F32)], axis=1)


def _full(shape):
    return pl.BlockSpec(shape, lambda i: (0, 0))


def _rows(shape):
    return pl.BlockSpec(shape, lambda i: (i, 0))


def _call(body, in_specs, out_shape, out_specs):
    return pl.pallas_call(
        body,
        grid=(_GRID,),
        in_specs=in_specs,
        out_shape=out_shape,
        out_specs=out_specs,
    )


def _edge_conv(x, idx_flat, wa_top, wa_bot, ba, wb, bb):
    f = x.shape[1]
    a, b = _call(
        _tab_body,
        [_rows((_BN, f)), _full((f, 64)), _full((1, 64)), _full((f, 64))],
        (jax.ShapeDtypeStruct((_NPAD, 64), _F32),
         jax.ShapeDtypeStruct((_NPAD, 128), _F32)),
        (_rows((_BN, 64)), _rows((_BN, 128))),
    )(x, wa_top - wa_bot, ba[None], wa_bot)
    bg = jnp.reshape(_sc_gather(b, idx_flat), (_K, _NPAD, 128))
    out = _call(
        _conv_body,
        [_rows((_BN, 64)),
         pl.BlockSpec((_K, _BN, 128), lambda i: (0, i, 0)),
         _full((64, 64)), _full((1, 64))],
        jax.ShapeDtypeStruct((_NPAD, 64), _F32),
        _rows((_BN, 64)),
    )(a, bg, wb, bb[None])
    return out


def kernel(points, normals, distances, w1a, b1a, w1b, b1b, w2a, b2a, w2b, b2b,
           w3a, b3a, w3b, b3b, w4, b4, w5, b5):
    xq = jnp.zeros((_NPAD, 8), _F32).at[:_N, :3].set(points)
    xt = xq.T

    nbrs = _call(
        _knn_body,
        [_rows((_BQ, 8)), _full((8, _NPAD))],
        jax.ShapeDtypeStruct((_NPAD, _K), jnp.int32),
        _rows((_BQ, _K)),
    )(xq, xt)

    idx_flat = jnp.reshape(nbrs.T, (-1,))

    pad8 = lambda w: jnp.zeros((8, 64), _F32).at[:3].set(w)
    f1 = _edge_conv(xq, idx_flat, pad8(w1a[:3]), pad8(w1a[3:]), b1a, w1b, b1b)
    f2 = _edge_conv(f1, idx_flat, w2a[:64], w2a[64:], b2a, w2b, b2b)
    f3 = _edge_conv(f2, idx_flat, w3a[:64], w3a[64:], b3a, w3b, b3b)

    w5p = jnp.zeros((256, 8), _F32).at[:, :3].set(w5)
    b5p = jnp.zeros((1, 8), _F32).at[:, :3].set(b5[None])
    pts = _call(
        _mlp_body,
        [_rows((_BN, 64)), _rows((_BN, 64)), _rows((_BN, 64)), _rows((_BN, 8)),
         _full((192, 256)), _full((1, 256)), _full((256, 8)), _full((1, 8))],
        jax.ShapeDtypeStruct((_NPAD, 8), _F32),
        _rows((_BN, 8)),
    )(f1, f2, f3, xq, w4, b4[None], w5p, b5p)

    nt = jnp.zeros((8, 8), _F32).at[:3].set(normals.T)
    dv = distances[None, :]
    stats = _call(
        _stats_body,
        [_rows((_BN, 8)), _full((8, 8)), _full((1, 8))],
        jax.ShapeDtypeStruct((16, _P), _F32),
        _full((16, _P)),
    )(pts, nt, dv)

    proj = _call(
        _proj_body,
        [_rows((_BN, 8)), _full((16, _P)), _full((8, 8)), _full((1, 8))],
        jax.ShapeDtypeStruct((_NPAD, 8), _F32),
        _rows((_BN, 8)),
    )(pts, stats, nt, dv)
    return proj[:_N, :3]


# speedup vs baseline: 5.3863x; 1.0937x over previous
import functools

import jax
import jax.numpy as jnp
from jax import lax
from jax.experimental import pallas as pl
from jax.experimental.pallas import tpu as pltpu
from jax.experimental.pallas import tpu_sc as plsc

_N = 10000
_K = 16
_P = 8
_THR = 0.05
_NPAD = 10240
_BQ = 512
_BN = 256
_GRID = _NPAD // _BN
_F32 = jnp.float32

_NC = 2
_NS = 16
_NW = _NC * _NS
_GB = _K * _NPAD
_BPW = _GB // _NW
_CH = 640


def _knn_body(xq_ref, xt_ref, nbr_ref):
    i = pl.program_id(0)
    xq = xq_ref[...]
    xt = xt_ref[...]
    sqq = jnp.sum(xq * xq, axis=1, keepdims=True)
    sqt = jnp.sum(xt * xt, axis=0, keepdims=True)
    mm = jnp.dot(xq, xt, preferred_element_type=_F32)
    d = sqq + sqt - 2.0 * mm
    col = jax.lax.broadcasted_iota(jnp.int32, d.shape, 1)
    row = jax.lax.broadcasted_iota(jnp.int32, d.shape, 0) + i * _BQ
    inf = _F32(jnp.inf)
    d = jnp.where((col == row) | (col >= _N), inf, d)
    big = jnp.int32(2 ** 30)
    idxs = []
    for _ in range(_K):
        m = jnp.min(d, axis=1, keepdims=True)
        am = jnp.min(jnp.where(d == m, col, big), axis=1, keepdims=True)
        idxs.append(am)
        d = jnp.where(col == am, inf, d)
    nbr_ref[...] = jnp.concatenate(idxs, axis=1)


def _tab_body(x_ref, wa_ref, ba_ref, wb_ref, a_ref, b_ref):
    x = x_ref[...]
    a_ref[...] = jnp.dot(x, wa_ref[...], preferred_element_type=_F32) + ba_ref[...]
    bt = jnp.dot(x, wb_ref[...], preferred_element_type=_F32)
    b_ref[...] = jnp.concatenate([bt, jnp.zeros_like(bt)], axis=1)


def _sc_gather_body(tab_hbm, idx_hbm, out_hbm, idx_v, rows_v, sem):
    wid = lax.axis_index("s") * _NC + lax.axis_index("c")
    base = wid * _BPW

    def step(i, carry):
        off = base + i * _CH
        pltpu.sync_copy(idx_hbm.at[pl.ds(off, _CH)], idx_v)
        pltpu.async_copy(tab_hbm.at[idx_v], rows_v, sem).wait()
        pltpu.sync_copy(rows_v, out_hbm.at[pl.ds(off, _CH)])
        return carry

    lax.fori_loop(0, _BPW // _CH, step, 0)


def _sc_gather(btab, idx_flat):
    mesh = plsc.VectorSubcoreMesh(core_axis_name="c", subcore_axis_name="s")
    call = functools.partial(
        pl.kernel,
        mesh=mesh,
        out_type=jax.ShapeDtypeStruct((_GB, 128), _F32),
        scratch_types=[
            pltpu.VMEM((_CH,), jnp.int32),
            pltpu.VMEM((_CH, 128), _F32),
            pltpu.SemaphoreType.DMA,
        ],
    )(_sc_gather_body)
    return call(btab, idx_flat)


def _conv_body(a_ref, bg_ref, wb_ref, bb_ref, f_ref):
    a = a_ref[...]
    wb = wb_ref[...]
    acc = None
    for j in range(_K):
        bj = bg_ref[j][:, :64]
        h = jnp.dot(jnp.maximum(a + bj, 0.0), wb, preferred_element_type=_F32)
        acc = h if acc is None else jnp.maximum(acc, h)
    f_ref[...] = acc + bb_ref[...]


def _mlp_body(f1_ref, f2_ref, f3_ref, p_ref, w4_ref, b4_ref, w5_ref, b5_ref, o_ref):
    feat = jnp.concatenate([f1_ref[...], f2_ref[...], f3_ref[...]], axis=1)
    h = jnp.maximum(jnp.dot(feat, w4_ref[...], preferred_element_type=_F32) + b4_ref[...], 0.0)
    r = jnp.dot(h, w5_ref[...], preferred_element_type=_F32) + b5_ref[...]
    o_ref[...] = p_ref[...] + r


def _stats_body(pts_ref, nt_ref, dv_ref, s_ref):
    i = pl.program_id(0)
    pts = pts_ref[...]
    pd = jnp.dot(pts, nt_ref[...], preferred_element_type=_F32) - dv_ref[...]
    rowid = jax.lax.broadcasted_iota(jnp.int32, pd.shape, 0) + i * _BN
    mask = jnp.where((jnp.abs(pd) < _THR) & (rowid < _N), 1.0, 0.0)
    x = pts[:, 0:1]
    y = pts[:, 1:2]
    z = pts[:, 2:3]
    rows = [
        jnp.sum(mask, axis=0, keepdims=True),
        jnp.sum(mask * x, axis=0, keepdims=True),
        jnp.sum(mask * y, axis=0, keepdims=True),
        jnp.sum(mask * z, axis=0, keepdims=True),
        jnp.sum(mask * x * x, axis=0, keepdims=True),
        jnp.sum(mask * x * y, axis=0, keepdims=True),
        jnp.sum(mask * x * z, axis=0, keepdims=True),
        jnp.sum(mask * y * y, axis=0, keepdims=True),
        jnp.sum(mask * y * z, axis=0, keepdims=True),
        jnp.sum(mask * z * z, axis=0, keepdims=True),
    ]
    rows += [jnp.zeros((1, _P), _F32)] * 6
    contrib = jnp.concatenate(rows, axis=0)
    prev = jnp.where(i == 0, jnp.zeros_like(contrib), s_ref[...])
    s_ref[...] = prev + contrib


def _eig_smallest(cov, nrows):
    a = [[cov[0], cov[1], cov[2]],
         [cov[1], cov[3], cov[4]],
         [cov[2], cov[4], cov[5]]]
    one = jnp.ones((1, _P), _F32)
    zero = jnp.zeros((1, _P), _F32)
    v = [[one, zero, zero], [zero, one, zero], [zero, zero, one]]
    for _ in range(8):
        for (p, q) in ((0, 1), (0, 2), (1, 2)):
            apq = a[p][q]
            small = jnp.abs(apq) < 1e-30
            apq_s = jnp.where(small, 1.0, apq)
            tau = (a[q][q] - a[p][p]) / (2.0 * apq_s)
            t = jnp.sign(tau) / (jnp.abs(tau) + jnp.sqrt(1.0 + tau * tau))
            t = jnp.where(jnp.sign(tau) == 0.0, 1.0 / (jnp.abs(tau) + jnp.sqrt(1.0 + tau * tau)), t)
            t = jnp.where(small, 0.0, t)
            c = 1.0 / jnp.sqrt(1.0 + t * t)
            s = t * c
            app = a[p][p] - t * apq
            aqq = a[q][q] + t * apq
            k = 3 - p - q
            akp = c * a[k][p] - s * a[k][q]
            akq = s * a[k][p] + c * a[k][q]
            a[p][p] = app
            a[q][q] = aqq
            a[p][q] = zero
            a[q][p] = zero
            a[k][p] = akp
            a[p][k] = akp
            a[k][q] = akq
            a[q][k] = akq
            for r in range(3):
                vrp = c * v[r][p] - s * v[r][q]
                vrq = s * v[r][p] + c * v[r][q]
                v[r][p] = vrp
                v[r][q] = vrq
    l0, l1, l2 = a[0][0], a[1][1], a[2][2]
    is0 = (l0 <= l1) & (l0 <= l2)
    is1 = jnp.logical_not(l0 <= l1) & (l1 <= l2)
    rn = [jnp.where(is0, v[r][0], jnp.where(is1, v[r][1], v[r][2])) for r in range(3)]
    dt = rn[0] * nrows[0] + rn[1] * nrows[1] + rn[2] * nrows[2]
    sgn = jnp.where(dt < 0.0, -1.0, 1.0)
    return [rn[r] * sgn for r in range(3)]


def _proj_body(pts_ref, s_ref, nt_ref, dv_ref, o_ref):
    pts = pts_ref[...]
    st = s_ref[...]
    cnt = st[0:1, :]
    safe = jnp.maximum(cnt, 1.0)
    cx = st[1:2, :] / safe
    cy = st[2:3, :] / safe
    cz = st[3:4, :] / safe
    cov = [st[4:5, :] - cnt * cx * cx,
           st[5:6, :] - cnt * cx * cy,
           st[6:7, :] - cnt * cx * cz,
           st[7:8, :] - cnt * cy * cy,
           st[8:9, :] - cnt * cy * cz,
           st[9:10, :] - cnt * cz * cz]
    nt = nt_ref[...]
    nrows = [nt[0:1, :], nt[1:2, :], nt[2:3, :]]
    rn = _eig_smallest(cov, nrows)
    rd = cx * rn[0] + cy * rn[1] + cz * rn[2]
    valid = jnp.where(cnt >= 3.0, 1.0, 0.0)

    pd = jnp.dot(pts, nt, preferred_element_type=_F32) - dv_ref[...]
    mask = jnp.where(jnp.abs(pd) < _THR, 1.0, 0.0) * valid

    px = pts[:, 0:1]
    py = pts[:, 1:2]
    pz = pts[:, 2:3]
    for p in range(_P):
        w = mask[:, p:p + 1]
        rx = rn[0][0:1, p:p + 1]
        ry = rn[1][0:1, p:p + 1]
        rz = rn[2][0:1, p:p + 1]
        dot = px * rx + py * ry + pz * rz
        adj = w * (dot - rd[0:1, p:p + 1])
        px = px - adj * rx
        py = py - adj * ry
        pz = pz - adj * rz
    o_ref[...] = jnp.concatenate([px, py, pz, jnp.zeros((px.shape[0], 5), _F32)], axis=1)


def _full(shape):
    return pl.BlockSpec(shape, lambda i: (0, 0))


def _rows(shape):
    return pl.BlockSpec(shape, lambda i: (i, 0))


def _call(body, in_specs, out_shape, out_specs, grid=_GRID):
    return pl.pallas_call(
        body,
        grid=(grid,),
        in_specs=in_specs,
        out_shape=out_shape,
        out_specs=out_specs,
    )


def _edge_conv(x, idx_flat, wa_top, wa_bot, ba, wb, bb):
    f = x.shape[1]
    a, b = _call(
        _tab_body,
        [_rows((_BN, f)), _full((f, 64)), _full((1, 64)), _full((f, 64))],
        (jax.ShapeDtypeStruct((_NPAD, 64), _F32),
         jax.ShapeDtypeStruct((_NPAD, 128), _F32)),
        (_rows((_BN, 64)), _rows((_BN, 128))),
    )(x, wa_top - wa_bot, ba[None], wa_bot)
    bg = jnp.reshape(_sc_gather(b, idx_flat), (_K, _NPAD, 128))
    out = _call(
        _conv_body,
        [_rows((_BN, 64)),
         pl.BlockSpec((_K, _BN, 128), lambda i: (0, i, 0)),
         _full((64, 64)), _full((1, 64))],
        jax.ShapeDtypeStruct((_NPAD, 64), _F32),
        _rows((_BN, 64)),
    )(a, bg, wb, bb[None])
    return out


def kernel(points, normals, distances, w1a, b1a, w1b, b1b, w2a, b2a, w2b, b2b,
           w3a, b3a, w3b, b3b, w4, b4, w5, b5):
    xq = jnp.zeros((_NPAD, 8), _F32).at[:_N, :3].set(points)
    xt = xq.T

    nbrs = _call(
        _knn_body,
        [_rows((_BQ, 8)), _full((8, _NPAD))],
        jax.ShapeDtypeStruct((_NPAD, _K), jnp.int32),
        _rows((_BQ, _K)),
        grid=_NPAD // _BQ,
    )(xq, xt)

    idx_flat = jnp.reshape(nbrs.T, (-1,))

    pad8 = lambda w: jnp.zeros((8, 64), _F32).at[:3].set(w)
    f1 = _edge_conv(xq, idx_flat, pad8(w1a[:3]), pad8(w1a[3:]), b1a, w1b, b1b)
    f2 = _edge_conv(f1, idx_flat, w2a[:64], w2a[64:], b2a, w2b, b2b)
    f3 = _edge_conv(f2, idx_flat, w3a[:64], w3a[64:], b3a, w3b, b3b)

    w5p = jnp.zeros((256, 8), _F32).at[:, :3].set(w5)
    b5p = jnp.zeros((1, 8), _F32).at[:, :3].set(b5[None])
    pts = _call(
        _mlp_body,
        [_rows((_BN, 64)), _rows((_BN, 64)), _rows((_BN, 64)), _rows((_BN, 8)),
         _full((192, 256)), _full((1, 256)), _full((256, 8)), _full((1, 8))],
        jax.ShapeDtypeStruct((_NPAD, 8), _F32),
        _rows((_BN, 8)),
    )(f1, f2, f3, xq, w4, b4[None], w5p, b5p)

    nt = jnp.zeros((8, 8), _F32).at[:3].set(normals.T)
    dv = distances[None, :]
    stats = _call(
        _stats_body,
        [_rows((_BN, 8)), _full((8, 8)), _full((1, 8))],
        jax.ShapeDtypeStruct((16, _P), _F32),
        _full((16, _P)),
    )(pts, nt, dv)

    proj = _call(
        _proj_body,
        [_rows((_BN, 8)), _full((16, _P)), _full((8, 8)), _full((1, 8))],
        jax.ShapeDtypeStruct((_NPAD, 8), _F32),
        _rows((_BN, 8)),
    )(pts, stats, nt, dv)
    return proj[:_N, :3]


# speedup vs baseline: 5.3907x; 1.0008x over previous
import functools

import jax
import jax.numpy as jnp
from jax import lax
from jax.experimental import pallas as pl
from jax.experimental.pallas import tpu as pltpu
from jax.experimental.pallas import tpu_sc as plsc

_N = 10000
_K = 16
_P = 8
_THR = 0.05
_NPAD = 10240
_BQ = 512
_BN = 256
_GRID = _NPAD // _BN
_F32 = jnp.float32

_NC = 2
_NS = 16
_NW = _NC * _NS
_GB = _K * _NPAD
_BPW = _GB // _NW
_CH = 320


def _knn_body(xq_ref, xt_ref, nbr_ref):
    i = pl.program_id(0)
    xq = xq_ref[...]
    xt = xt_ref[...]
    sqq = jnp.sum(xq * xq, axis=1, keepdims=True)
    sqt = jnp.sum(xt * xt, axis=0, keepdims=True)
    mm = jnp.dot(xq, xt, preferred_element_type=_F32)
    d = sqq + sqt - 2.0 * mm
    col = jax.lax.broadcasted_iota(jnp.int32, d.shape, 1)
    row = jax.lax.broadcasted_iota(jnp.int32, d.shape, 0) + i * _BQ
    inf = _F32(jnp.inf)
    d = jnp.where((col == row) | (col >= _N), inf, d)
    big = jnp.int32(2 ** 30)
    idxs = []
    for _ in range(_K):
        m = jnp.min(d, axis=1, keepdims=True)
        am = jnp.min(jnp.where(d == m, col, big), axis=1, keepdims=True)
        idxs.append(am)
        d = jnp.where(col == am, inf, d)
    nbr_ref[...] = jnp.concatenate(idxs, axis=1)


def _tab_body(x_ref, wa_ref, ba_ref, wb_ref, a_ref, b_ref):
    x = x_ref[...]
    a_ref[...] = jnp.dot(x, wa_ref[...], preferred_element_type=_F32) + ba_ref[...]
    bt = jnp.dot(x, wb_ref[...], preferred_element_type=_F32)
    b_ref[...] = jnp.concatenate([bt, jnp.zeros_like(bt)], axis=1)


def _sc_gather_body(tab_hbm, idx_hbm, out_hbm,
                    idx0, idx1, rows0, rows1, gs0, gs1, ws0, ws1):
    wid = lax.axis_index("s") * _NC + lax.axis_index("c")
    base = wid * _BPW
    nch = _BPW // _CH
    idx_v = (idx0, idx1)
    rows = (rows0, rows1)
    gsem = (gs0, gs1)
    wsem = (ws0, ws1)

    gh = [None, None]
    wh = [None, None]
    pltpu.sync_copy(idx_hbm.at[pl.ds(base, _CH)], idx_v[0])
    gh[0] = pltpu.async_copy(tab_hbm.at[idx_v[0]], rows[0], gsem[0])
    for i in range(nch):
        b = i % 2
        if i + 1 < nch:
            nb = (i + 1) % 2
            if wh[nb] is not None:
                wh[nb].wait()
            pltpu.sync_copy(idx_hbm.at[pl.ds(base + (i + 1) * _CH, _CH)],
                            idx_v[nb])
            gh[nb] = pltpu.async_copy(tab_hbm.at[idx_v[nb]], rows[nb],
                                      gsem[nb])
        gh[b].wait()
        wh[b] = pltpu.async_copy(rows[b], out_hbm.at[pl.ds(base + i * _CH, _CH)],
                                 wsem[b])
    for b in range(2):
        if wh[b] is not None:
            wh[b].wait()


def _sc_gather(btab, idx_flat):
    mesh = plsc.VectorSubcoreMesh(core_axis_name="c", subcore_axis_name="s")
    call = functools.partial(
        pl.kernel,
        mesh=mesh,
        out_type=jax.ShapeDtypeStruct((_GB, 128), _F32),
        scratch_types=[
            pltpu.VMEM((_CH,), jnp.int32),
            pltpu.VMEM((_CH,), jnp.int32),
            pltpu.VMEM((_CH, 128), _F32),
            pltpu.VMEM((_CH, 128), _F32),
            pltpu.SemaphoreType.DMA,
            pltpu.SemaphoreType.DMA,
            pltpu.SemaphoreType.DMA,
            pltpu.SemaphoreType.DMA,
        ],
    )(_sc_gather_body)
    return call(btab, idx_flat)


def _conv_body(a_ref, bg_ref, wb_ref, bb_ref, f_ref):
    a = a_ref[...]
    wb = wb_ref[...]
    acc = None
    for j in range(_K):
        bj = bg_ref[j][:, :64]
        h = jnp.dot(jnp.maximum(a + bj, 0.0), wb, preferred_element_type=_F32)
        acc = h if acc is None else jnp.maximum(acc, h)
    f_ref[...] = acc + bb_ref[...]


def _mlp_body(f1_ref, f2_ref, f3_ref, p_ref, w4_ref, b4_ref, w5_ref, b5_ref, o_ref):
    feat = jnp.concatenate([f1_ref[...], f2_ref[...], f3_ref[...]], axis=1)
    h = jnp.maximum(jnp.dot(feat, w4_ref[...], preferred_element_type=_F32) + b4_ref[...], 0.0)
    r = jnp.dot(h, w5_ref[...], preferred_element_type=_F32) + b5_ref[...]
    o_ref[...] = p_ref[...] + r


def _stats_body(pts_ref, nt_ref, dv_ref, s_ref):
    i = pl.program_id(0)
    pts = pts_ref[...]
    pd = jnp.dot(pts, nt_ref[...], preferred_element_type=_F32) - dv_ref[...]
    rowid = jax.lax.broadcasted_iota(jnp.int32, pd.shape, 0) + i * _BN
    mask = jnp.where((jnp.abs(pd) < _THR) & (rowid < _N), 1.0, 0.0)
    x = pts[:, 0:1]
    y = pts[:, 1:2]
    z = pts[:, 2:3]
    rows = [
        jnp.sum(mask, axis=0, keepdims=True),
        jnp.sum(mask * x, axis=0, keepdims=True),
        jnp.sum(mask * y, axis=0, keepdims=True),
        jnp.sum(mask * z, axis=0, keepdims=True),
        jnp.sum(mask * x * x, axis=0, keepdims=True),
        jnp.sum(mask * x * y, axis=0, keepdims=True),
        jnp.sum(mask * x * z, axis=0, keepdims=True),
        jnp.sum(mask * y * y, axis=0, keepdims=True),
        jnp.sum(mask * y * z, axis=0, keepdims=True),
        jnp.sum(mask * z * z, axis=0, keepdims=True),
    ]
    rows += [jnp.zeros((1, _P), _F32)] * 6
    contrib = jnp.concatenate(rows, axis=0)
    prev = jnp.where(i == 0, jnp.zeros_like(contrib), s_ref[...])
    s_ref[...] = prev + contrib


def _eig_smallest(cov, nrows):
    a = [[cov[0], cov[1], cov[2]],
         [cov[1], cov[3], cov[4]],
         [cov[2], cov[4], cov[5]]]
    one = jnp.ones((1, _P), _F32)
    zero = jnp.zeros((1, _P), _F32)
    v = [[one, zero, zero], [zero, one, zero], [zero, zero, one]]
    for _ in range(8):
        for (p, q) in ((0, 1), (0, 2), (1, 2)):
            apq = a[p][q]
            small = jnp.abs(apq) < 1e-30
            apq_s = jnp.where(small, 1.0, apq)
            tau = (a[q][q] - a[p][p]) / (2.0 * apq_s)
            t = jnp.sign(tau) / (jnp.abs(tau) + jnp.sqrt(1.0 + tau * tau))
            t = jnp.where(jnp.sign(tau) == 0.0, 1.0 / (jnp.abs(tau) + jnp.sqrt(1.0 + tau * tau)), t)
            t = jnp.where(small, 0.0, t)
            c = 1.0 / jnp.sqrt(1.0 + t * t)
            s = t * c
            app = a[p][p] - t * apq
            aqq = a[q][q] + t * apq
            k = 3 - p - q
            akp = c * a[k][p] - s * a[k][q]
            akq = s * a[k][p] + c * a[k][q]
            a[p][p] = app
            a[q][q] = aqq
            a[p][q] = zero
            a[q][p] = zero
            a[k][p] = akp
            a[p][k] = akp
            a[k][q] = akq
            a[q][k] = akq
            for r in range(3):
                vrp = c * v[r][p] - s * v[r][q]
                vrq = s * v[r][p] + c * v[r][q]
                v[r][p] = vrp
                v[r][q] = vrq
    l0, l1, l2 = a[0][0], a[1][1], a[2][2]
    is0 = (l0 <= l1) & (l0 <= l2)
    is1 = jnp.logical_not(l0 <= l1) & (l1 <= l2)
    rn = [jnp.where(is0, v[r][0], jnp.where(is1, v[r][1], v[r][2])) for r in range(3)]
    dt = rn[0] * nrows[0] + rn[1] * nrows[1] + rn[2] * nrows[2]
    sgn = jnp.where(dt < 0.0, -1.0, 1.0)
    return [rn[r] * sgn for r in range(3)]


def _proj_body(pts_ref, s_ref, nt_ref, dv_ref, o_ref):
    pts = pts_ref[...]
    st = s_ref[...]
    cnt = st[0:1, :]
    safe = jnp.maximum(cnt, 1.0)
    cx = st[1:2, :] / safe
    cy = st[2:3, :] / safe
    cz = st[3:4, :] / safe
    cov = [st[4:5, :] - cnt * cx * cx,
           st[5:6, :] - cnt * cx * cy,
           st[6:7, :] - cnt * cx * cz,
           st[7:8, :] - cnt * cy * cy,
           st[8:9, :] - cnt * cy * cz,
           st[9:10, :] - cnt * cz * cz]
    nt = nt_ref[...]
    nrows = [nt[0:1, :], nt[1:2, :], nt[2:3, :]]
    rn = _eig_smallest(cov, nrows)
    rd = cx * rn[0] + cy * rn[1] + cz * rn[2]
    valid = jnp.where(cnt >= 3.0, 1.0, 0.0)

    pd = jnp.dot(pts, nt, preferred_element_type=_F32) - dv_ref[...]
    mask = jnp.where(jnp.abs(pd) < _THR, 1.0, 0.0) * valid

    px = pts[:, 0:1]
    py = pts[:, 1:2]
    pz = pts[:, 2:3]
    for p in range(_P):
        w = mask[:, p:p + 1]
        rx = rn[0][0:1, p:p + 1]
        ry = rn[1][0:1, p:p + 1]
        rz = rn[2][0:1, p:p + 1]
        dot = px * rx + py * ry + pz * rz
        adj = w * (dot - rd[0:1, p:p + 1])
        px = px - adj * rx
        py = py - adj * ry
        pz = pz - adj * rz
    o_ref[...] = jnp.concatenate([px, py, pz, jnp.zeros((px.shape[0], 5), _F32)], axis=1)


def _full(shape):
    return pl.BlockSpec(shape, lambda i: (0, 0))


def _rows(shape):
    return pl.BlockSpec(shape, lambda i: (i, 0))


def _call(body, in_specs, out_shape, out_specs, grid=_GRID):
    return pl.pallas_call(
        body,
        grid=(grid,),
        in_specs=in_specs,
        out_shape=out_shape,
        out_specs=out_specs,
    )


def _edge_conv(x, idx_flat, wa_top, wa_bot, ba, wb, bb):
    f = x.shape[1]
    a, b = _call(
        _tab_body,
        [_rows((_BN, f)), _full((f, 64)), _full((1, 64)), _full((f, 64))],
        (jax.ShapeDtypeStruct((_NPAD, 64), _F32),
         jax.ShapeDtypeStruct((_NPAD, 128), _F32)),
        (_rows((_BN, 64)), _rows((_BN, 128))),
    )(x, wa_top - wa_bot, ba[None], wa_bot)
    bg = jnp.reshape(_sc_gather(b, idx_flat), (_K, _NPAD, 128))
    out = _call(
        _conv_body,
        [_rows((_BN, 64)),
         pl.BlockSpec((_K, _BN, 128), lambda i: (0, i, 0)),
         _full((64, 64)), _full((1, 64))],
        jax.ShapeDtypeStruct((_NPAD, 64), _F32),
        _rows((_BN, 64)),
    )(a, bg, wb, bb[None])
    return out


def kernel(points, normals, distances, w1a, b1a, w1b, b1b, w2a, b2a, w2b, b2b,
           w3a, b3a, w3b, b3b, w4, b4, w5, b5):
    xq = jnp.zeros((_NPAD, 8), _F32).at[:_N, :3].set(points)
    xt = xq.T

    nbrs = _call(
        _knn_body,
        [_rows((_BQ, 8)), _full((8, _NPAD))],
        jax.ShapeDtypeStruct((_NPAD, _K), jnp.int32),
        _rows((_BQ, _K)),
        grid=_NPAD // _BQ,
    )(xq, xt)

    idx_flat = jnp.reshape(nbrs.T, (-1,))

    pad8 = lambda w: jnp.zeros((8, 64), _F32).at[:3].set(w)
    f1 = _edge_conv(xq, idx_flat, pad8(w1a[:3]), pad8(w1a[3:]), b1a, w1b, b1b)
    f2 = _edge_conv(f1, idx_flat, w2a[:64], w2a[64:], b2a, w2b, b2b)
    f3 = _edge_conv(f2, idx_flat, w3a[:64], w3a[64:], b3a, w3b, b3b)

    w5p = jnp.zeros((256, 8), _F32).at[:, :3].set(w5)
    b5p = jnp.zeros((1, 8), _F32).at[:, :3].set(b5[None])
    pts = _call(
        _mlp_body,
        [_rows((_BN, 64)), _rows((_BN, 64)), _rows((_BN, 64)), _rows((_BN, 8)),
         _full((192, 256)), _full((1, 256)), _full((256, 8)), _full((1, 8))],
        jax.ShapeDtypeStruct((_NPAD, 8), _F32),
        _rows((_BN, 8)),
    )(f1, f2, f3, xq, w4, b4[None], w5p, b5p)

    nt = jnp.zeros((8, 8), _F32).at[:3].set(normals.T)
    dv = distances[None, :]
    stats = _call(
        _stats_body,
        [_rows((_BN, 8)), _full((8, 8)), _full((1, 8))],
        jax.ShapeDtypeStruct((16, _P), _F32),
        _full((16, _P)),
    )(pts, nt, dv)

    proj = _call(
        _proj_body,
        [_rows((_BN, 8)), _full((16, _P)), _full((8, 8)), _full((1, 8))],
        jax.ShapeDtypeStruct((_NPAD, 8), _F32),
        _rows((_BN, 8)),
    )(pts, stats, nt, dv)
    return proj[:_N, :3]
